# baseline (speedup 1.0000x reference)
import functools

import jax
import jax.numpy as jnp
from jax import lax
from jax.experimental import pallas as pl
from jax.experimental.pallas import tpu as pltpu
from jax.experimental.pallas import tpu_sc as plsc

N = 10000
E = 320000
D = 128
HD = D // 2
SCALE = 1.0
EPS = 1e-12

NC, NS = 2, 16
CHUNK = 128
N_CHUNKS = 160
EDGES_PER_TILE = CHUNK * N_CHUNKS
E_PAD = NS * EDGES_PER_TILE
CHUNK_ROWS = E_PAD // CHUNK
N_PAD = 10240
ROWS_PER_TILE = N_PAD // NS
NBUF = 3
IB = 8
N_BATCH = N_CHUNKS // IB
BR = 1000


def _sc_neighbor_stats(x2, src2, dst):
    mesh = plsc.VectorSubcoreMesh(core_axis_name="c", subcore_axis_name="s")
    out_type = [
        jax.ShapeDtypeStruct((NC, N_PAD, HD), jnp.float32),
        jax.ShapeDtypeStruct((NC, N_PAD), jnp.float32),
    ]
    scratch = [
        pltpu.VMEM_SHARED((N_PAD, HD), jnp.float32),
        pltpu.VMEM_SHARED((N_PAD, HD), jnp.float32),
        pltpu.VMEM_SHARED((N_PAD,), jnp.float32),
        pltpu.VMEM((NBUF, CHUNK, HD), jnp.float32),
        pltpu.VMEM((CHUNK, HD), jnp.float32),
        pltpu.VMEM((2, IB, CHUNK), jnp.int32),
        pltpu.VMEM((2, IB, CHUNK), jnp.int32),
        pltpu.VMEM((CHUNK,), jnp.float32),
        pltpu.VMEM((ROWS_PER_TILE,), jnp.float32),
        pltpu.SMEM((ROWS_PER_TILE,), jnp.float32),
    ] + [pltpu.SemaphoreType.DMA] * (NBUF + 2)

    @functools.partial(pl.kernel, out_type=out_type, mesh=mesh,
                       scratch_types=scratch,
                       compiler_params=pltpu.CompilerParams(
                           use_tc_tiling_on_sc=False))
    def k(x2_h, src_h, dst_h, s2_out, deg_out,
          acc, acc2, dacc, rows, xbuf, sidx, didx, ones, dz, dzs,
          gsem0, gsem1, gsem2, dsem, isem):
        gsem = [gsem0, gsem1, gsem2]
        c = lax.axis_index("c")
        s = lax.axis_index("s")
        row0 = s * ROWS_PER_TILE
        crow0 = s * N_CHUNKS

        zv = jnp.zeros((16,), jnp.float32)
        ov = jnp.ones((16,), jnp.float32)

        def zrow(i, carry):
            for j in range(HD // 16):
                rows[0, i, pl.ds(j * 16, 16)] = zv
            return carry
        lax.fori_loop(0, CHUNK, zrow, 0)
        for j in range(ROWS_PER_TILE // 16):
            dz[pl.ds(j * 16, 16)] = zv
        for j in range(CHUNK // 16):
            ones[pl.ds(j * 16, 16)] = ov

        def zacc(g, carry):
            r0 = row0 + g * CHUNK
            pltpu.sync_copy(rows.at[0], acc.at[pl.ds(r0, CHUNK), :])
            pltpu.sync_copy(rows.at[0], acc2.at[pl.ds(r0, CHUNK), :])
            return carry
        lax.fori_loop(0, ROWS_PER_TILE // CHUNK, zacc, 0)
        pltpu.sync_copy(dz, dacc.at[pl.ds(row0, ROWS_PER_TILE)])
        plsc.subcore_barrier()

        def src_slice(cidx, r0, n):
            return src_h.at[cidx, pl.ds(r0, n), :]

        def run_pass(table, target, cidx, with_deg):
            pltpu.sync_copy(src_slice(cidx, crow0, IB), sidx.at[0])
            pltpu.sync_copy(dst_h.at[pl.ds(crow0, IB), :], didx.at[0])
            gd = [None] * NBUF
            ipf = None
            for kk in range(NBUF - 1):
                gd[kk] = pltpu.async_copy(table.at[sidx.at[0, kk]],
                                          rows.at[kk], gsem[kk])
            deg_ds = [[] for _ in range(N_BATCH)]
            for kk in range(N_CHUNKS):
                b = kk % NBUF
                bt, j = divmod(kk, IB)
                pb = bt % 2
                if j == 0 and bt + 1 < N_BATCH:
                    if bt >= 1:
                        for dd in deg_ds[bt - 1]:
                            dd.wait()
                        deg_ds[bt - 1] = []
                    npb = (bt + 1) % 2
                    r0 = crow0 + (bt + 1) * IB
                    ipf = (pltpu.async_copy(src_slice(cidx, r0, IB),
                                            sidx.at[npb], isem),
                           pltpu.async_copy(dst_h.at[pl.ds(r0, IB), :],
                                            didx.at[npb], isem))
                nxt = kk + NBUF - 1
                if nxt < N_CHUNKS:
                    nbt, nj = divmod(nxt, IB)
                    if nj < NBUF - 1 and nbt > bt:
                        for d in ipf:
                            d.wait()
                        ipf = ()
                    nb = nxt % NBUF
                    gd[nb] = pltpu.async_copy(
                        table.at[sidx.at[nbt % 2, nj]], rows.at[nb],
                        gsem[nb])
                if with_deg:
                    deg_ds[bt].append(
                        pltpu.async_copy(ones, dacc.at[didx.at[pb, j]],
                                         dsem, add=True))
                gd[b].wait()
                pltpu.sync_copy(rows.at[b], target.at[didx.at[pb, j]],
                                add=True)
            for lst in deg_ds:
                for dd in lst:
                    dd.wait()

        run_pass(x2_h, acc, c, True)
        plsc.subcore_barrier()

        pltpu.sync_copy(dacc.at[pl.ds(row0, ROWS_PER_TILE)], dzs)

        def rchunk(g, carry):
            r0 = row0 + g * CHUNK
            pltpu.sync_copy(acc.at[pl.ds(r0, CHUNK), :], rows.at[0])
            pltpu.sync_copy(x2_h.at[pl.ds(c * N_PAD + r0, CHUNK), :], xbuf)

            def rbody(row, carry2):
                dval = dzs[g * CHUNK + row]
                bd = jnp.full((16,), dval, jnp.float32)
                dinv = 1.0 / jnp.maximum(bd, 1.0)
                for t in range(HD // 16):
                    m = rows[0, row, pl.ds(t * 16, 16)] * dinv
                    dlt = xbuf[row, pl.ds(t * 16, 16)] - m
                    rows[0, row, pl.ds(t * 16, 16)] = dlt * dlt
                return carry2
            lax.fori_loop(0, CHUNK, rbody, 0)
            pltpu.sync_copy(rows.at[0], acc.at[pl.ds(r0, CHUNK), :])
            return carry
        lax.fori_loop(0, ROWS_PER_TILE // CHUNK, rchunk, 0)
        plsc.subcore_barrier()

        run_pass(acc, acc2, 0 * c, False)
        plsc.subcore_barrier()

        pltpu.sync_copy(acc2.at[pl.ds(row0, ROWS_PER_TILE), :],
                        s2_out.at[c, pl.ds(row0, ROWS_PER_TILE), :])
        pltpu.sync_copy(dacc.at[pl.ds(row0, ROWS_PER_TILE)], dz)
        pltpu.sync_copy(dz, deg_out.at[c, pl.ds(row0, ROWS_PER_TILE)])

    return k(x2, src2, dst)


def _tc_final(s2, deg, x):
    def body(s_ref, d_ref, x_ref, o_ref):
        degc = jnp.maximum(d_ref[0], 1.0)
        var = jnp.concatenate([s_ref[0], s_ref[1]], axis=-1) / degc
        std = jnp.sqrt(var + EPS)
        xv = x_ref[...]
        o = SCALE * xv / std
        o_ref[...] = jnp.where(jnp.isinf(o), xv, o)

    return pl.pallas_call(
        body,
        grid=(N // BR,),
        in_specs=[
            pl.BlockSpec((2, BR, HD), lambda i: (0, i, 0)),
            pl.BlockSpec((2, BR, 1), lambda i: (0, i, 0)),
            pl.BlockSpec((BR, D), lambda i: (i, 0)),
        ],
        out_specs=pl.BlockSpec((BR, D), lambda i: (i, 0)),
        out_shape=jax.ShapeDtypeStruct((N, D), jnp.float32),
    )(s2, deg, x)


def kernel(x, edge_index):
    ei = edge_index.astype(jnp.int32)
    dst = ei[0]
    src = ei[1]
    pad = E_PAD - E
    pad_ids = lax.iota(jnp.int32, pad)
    src_p = jnp.concatenate([src, pad_ids % N]).reshape(CHUNK_ROWS, CHUNK)
    dst_p = jnp.concatenate([dst, N + pad_ids % (N_PAD - N)]
                            ).reshape(CHUNK_ROWS, CHUNK)
    src2 = jnp.stack([src_p, src_p + N_PAD])
    x2 = jnp.zeros((2, N_PAD, HD), jnp.float32)
    x2 = x2.at[0, :N].set(x[:, :HD]).at[1, :N].set(x[:, HD:])
    x2 = x2.reshape(2 * N_PAD, HD)

    s2, deg = _sc_neighbor_stats(x2, src2, dst_p)
    deg3 = deg.reshape(NC, N_PAD, 1)
    return _tc_final(s2, deg3, x)


# speedup vs baseline: 1.2954x; 1.2954x over previous
import functools

import jax
import jax.numpy as jnp
from jax import lax
from jax.experimental import pallas as pl
from jax.experimental.pallas import tpu as pltpu
from jax.experimental.pallas import tpu_sc as plsc

N = 10000
E = 320000
D = 128
SCALE = 1.0
EPS = 1e-12

NC, NS = 2, 16
NW = NC * NS
CHUNK = 80
N_CHUNKS = 128
EDGES_PER_TILE = CHUNK * N_CHUNKS
E_PAD = NW * EDGES_PER_TILE
CHUNK_ROWS = E_PAD // CHUNK
N_PAD = 10240
ROWS_PER_TILE = N_PAD // NS
NBUF = 4
IB = 8
N_BATCH = N_CHUNKS // IB
BR = 1000


def _sc_segment_sums(table, src, dst, compute_deg):
    mesh = plsc.VectorSubcoreMesh(core_axis_name="c", subcore_axis_name="s")
    out_type = [jax.ShapeDtypeStruct((NC, N_PAD, D), jnp.float32)]
    if compute_deg:
        out_type.append(jax.ShapeDtypeStruct((NC, N_PAD), jnp.float32))
    scratch = [
        pltpu.VMEM_SHARED((N_PAD, D), jnp.float32),
        pltpu.VMEM_SHARED((N_PAD,), jnp.float32),
        pltpu.VMEM((NBUF, CHUNK, D), jnp.float32),
        pltpu.VMEM((2, IB, CHUNK), jnp.int32),
        pltpu.VMEM((2, IB, CHUNK), jnp.int32),
        pltpu.VMEM((CHUNK,), jnp.float32),
        pltpu.VMEM((ROWS_PER_TILE,), jnp.float32),
    ] + [pltpu.SemaphoreType.DMA] * (NBUF + 2)

    @functools.partial(pl.kernel, out_type=out_type, mesh=mesh,
                       scratch_types=scratch)
    def k(table_h, src_h, dst_h, sum_out, *rest):
        if compute_deg:
            deg_out = rest[0]
            rest = rest[1:]
        (acc, dacc, rows, sidx, didx, ones, dz,
         gsem0, gsem1, gsem2, gsem3, dsem, isem) = rest
        gsem = [gsem0, gsem1, gsem2, gsem3]
        c = lax.axis_index("c")
        s = lax.axis_index("s")
        wid = c * NS + s
        row0 = s * ROWS_PER_TILE
        crow0 = wid * N_CHUNKS

        zv = jnp.zeros((16,), jnp.float32)
        ov = jnp.ones((16,), jnp.float32)

        pltpu.sync_copy(src_h.at[pl.ds(crow0, IB), :], sidx.at[0])
        pltpu.sync_copy(dst_h.at[pl.ds(crow0, IB), :], didx.at[0])

        def zrow(i, carry):
            for j in range(D // 16):
                rows[NBUF - 1, i, pl.ds(j * 16, 16)] = zv
            return carry
        lax.fori_loop(0, CHUNK, zrow, 0)
        if compute_deg:
            for j in range(ROWS_PER_TILE // 16):
                dz[pl.ds(j * 16, 16)] = zv
            for j in range(CHUNK // 16):
                ones[pl.ds(j * 16, 16)] = ov

        gd = [None] * NBUF
        ipf = None
        for kk in range(NBUF - 1):
            gd[kk] = pltpu.async_copy(table_h.at[sidx.at[0, kk]],
                                      rows.at[kk], gsem[kk])
        zds = []
        for g in range(ROWS_PER_TILE // CHUNK):
            zds.append(pltpu.async_copy(
                rows.at[NBUF - 1],
                acc.at[pl.ds(row0 + g * CHUNK, CHUNK), :], dsem))
        rem = ROWS_PER_TILE % CHUNK
        if rem:
            r0 = row0 + (ROWS_PER_TILE // CHUNK) * CHUNK
            zds.append(pltpu.async_copy(
                rows.at[NBUF - 1, pl.ds(0, rem), :],
                acc.at[pl.ds(r0, rem), :], dsem))
        if compute_deg:
            zds.append(pltpu.async_copy(
                dz, dacc.at[pl.ds(row0, ROWS_PER_TILE)], dsem))
        for d in zds:
            d.wait()
        plsc.subcore_barrier()

        deg_ds = [[] for _ in range(N_BATCH)]
        for kk in range(N_CHUNKS):
            b = kk % NBUF
            bt, j = divmod(kk, IB)
            pb = bt % 2
            if j == 0 and bt + 1 < N_BATCH:
                if bt >= 1:
                    for dd in deg_ds[bt - 1]:
                        dd.wait()
                    deg_ds[bt - 1] = []
                npb = (bt + 1) % 2
                r0 = crow0 + (bt + 1) * IB
                ipf = (pltpu.async_copy(src_h.at[pl.ds(r0, IB), :],
                                        sidx.at[npb], isem),
                       pltpu.async_copy(dst_h.at[pl.ds(r0, IB), :],
                                        didx.at[npb], isem))
            nxt = kk + NBUF - 1
            if nxt < N_CHUNKS:
                nbt, nj = divmod(nxt, IB)
                if nj < NBUF - 1 and nbt > bt:
                    for d in ipf:
                        d.wait()
                    ipf = ()
                nb = nxt % NBUF
                gd[nb] = pltpu.async_copy(
                    table_h.at[sidx.at[nbt % 2, nj]], rows.at[nb], gsem[nb])
            if compute_deg:
                deg_ds[bt].append(
                    pltpu.async_copy(ones, dacc.at[didx.at[pb, j]], dsem,
                                     add=True))
            gd[b].wait()
            pltpu.sync_copy(rows.at[b], acc.at[didx.at[pb, j]], add=True)
        for lst in deg_ds:
            for dd in lst:
                dd.wait()
        plsc.subcore_barrier()

        pltpu.sync_copy(acc.at[pl.ds(row0, ROWS_PER_TILE), :],
                        sum_out.at[c, pl.ds(row0, ROWS_PER_TILE), :])
        if compute_deg:
            pltpu.sync_copy(dacc.at[pl.ds(row0, ROWS_PER_TILE)],
                            deg_out.at[c, pl.ds(row0, ROWS_PER_TILE)])

    return k(table, src, dst)


def _tc_mean_r(s1, deg, x):
    def body(s_ref, d_ref, x_ref, r_ref):
        degc = jnp.maximum(d_ref[0] + d_ref[1], 1.0)
        mean = (s_ref[0] + s_ref[1]) / degc
        dlt = x_ref[...] - mean
        r_ref[...] = dlt * dlt

    return pl.pallas_call(
        body,
        grid=(N // BR,),
        in_specs=[
            pl.BlockSpec((2, BR, D), lambda i: (0, i, 0)),
            pl.BlockSpec((2, BR, 1), lambda i: (0, i, 0)),
            pl.BlockSpec((BR, D), lambda i: (i, 0)),
        ],
        out_specs=pl.BlockSpec((BR, D), lambda i: (i, 0)),
        out_shape=jax.ShapeDtypeStruct((N, D), jnp.float32),
    )(s1, deg, x)


def _tc_final(s2, deg, x):
    def body(s_ref, d_ref, x_ref, o_ref):
        degc = jnp.maximum(d_ref[0] + d_ref[1], 1.0)
        var = (s_ref[0] + s_ref[1]) / degc
        std = jnp.sqrt(var + EPS)
        xv = x_ref[...]
        o = SCALE * xv / std
        o_ref[...] = jnp.where(jnp.isinf(o), xv, o)

    return pl.pallas_call(
        body,
        grid=(N // BR,),
        in_specs=[
            pl.BlockSpec((2, BR, D), lambda i: (0, i, 0)),
            pl.BlockSpec((2, BR, 1), lambda i: (0, i, 0)),
            pl.BlockSpec((BR, D), lambda i: (i, 0)),
        ],
        out_specs=pl.BlockSpec((BR, D), lambda i: (i, 0)),
        out_shape=jax.ShapeDtypeStruct((N, D), jnp.float32),
    )(s2, deg, x)


def kernel(x, edge_index):
    ei = edge_index.astype(jnp.int32)
    dst = ei[0]
    src = ei[1]
    pad = E_PAD - E
    pad_ids = lax.iota(jnp.int32, pad)
    src_p = jnp.concatenate([src, pad_ids % N]).reshape(CHUNK_ROWS, CHUNK)
    dst_p = jnp.concatenate([dst, N + pad_ids % (N_PAD - N)]
                            ).reshape(CHUNK_ROWS, CHUNK)

    s1, deg = _sc_segment_sums(x, src_p, dst_p, compute_deg=True)
    deg3 = deg.reshape(NC, N_PAD, 1)
    r = _tc_mean_r(s1, deg3, x)
    (s2,) = _sc_segment_sums(r, src_p, dst_p, compute_deg=False)
    return _tc_final(s2, deg3, x)
